# baseline (speedup 1.0000x reference)
import math

import jax
import jax.numpy as jnp
import numpy as np
from jax.experimental import pallas as pl
from jax.experimental.pallas import tpu as pltpu

_BS = 512


def _body(freq_ref, phase_ref, x_ref, o_ref, pe_ref):
    b = pl.program_id(1)

    @pl.when(b == 0)
    def _compute_pe():
        s = pl.program_id(0)
        pos = (
            jax.lax.broadcasted_iota(jnp.int32, (_BS, 1), 0) + s * _BS
        ).astype(jnp.float32)
        arg = pos * freq_ref[...] + phase_ref[...]
        pe_ref[...] = jnp.sin(arg)

    o_ref[0, :, :] = x_ref[0, :, :] + pe_ref[...]


def kernel(x, abs_pe):
    B, S, D = x.shape
    div = np.exp(
        np.arange(0, D, 2).astype(np.float32) * (-math.log(10000.0) / D)
    )
    freq = jnp.asarray(np.repeat(div, 2).reshape(1, D))
    phase = jnp.asarray(
        np.tile(np.array([0.0, 0.5 * math.pi], np.float32), D // 2).reshape(1, D)
    )
    grid = (S // _BS, B)
    out = pl.pallas_call(
        _body,
        grid=grid,
        in_specs=[
            pl.BlockSpec((1, D), lambda s, b: (0, 0)),
            pl.BlockSpec((1, D), lambda s, b: (0, 0)),
            pl.BlockSpec((1, _BS, D), lambda s, b: (b, s, 0)),
        ],
        out_specs=pl.BlockSpec((1, _BS, D), lambda s, b: (b, s, 0)),
        out_shape=jax.ShapeDtypeStruct(x.shape, x.dtype),
        scratch_shapes=[pltpu.VMEM((_BS, D), jnp.float32)],
        compiler_params=pltpu.CompilerParams(
            dimension_semantics=("arbitrary", "arbitrary"),
        ),
    )(freq, phase, x)
    return out


# speedup vs baseline: 2.0336x; 2.0336x over previous
import jax
import jax.numpy as jnp
from jax.experimental import pallas as pl
from jax.experimental.pallas import tpu as pltpu

_BS = 1024


def _body(pe_ref, x_ref, o_ref):
    o_ref[0, :, :] = x_ref[0, :, :] + pe_ref[0, :, :]


def kernel(x, abs_pe):
    B, S, D = x.shape
    grid = (S // _BS, B)
    out = pl.pallas_call(
        _body,
        grid=grid,
        in_specs=[
            pl.BlockSpec((1, _BS, D), lambda s, b: (0, s, 0)),
            pl.BlockSpec((1, _BS, D), lambda s, b: (b, s, 0)),
        ],
        out_specs=pl.BlockSpec((1, _BS, D), lambda s, b: (b, s, 0)),
        out_shape=jax.ShapeDtypeStruct(x.shape, x.dtype),
        compiler_params=pltpu.CompilerParams(
            dimension_semantics=("arbitrary", "arbitrary"),
        ),
    )(abs_pe, x)
    return out


# speedup vs baseline: 2.0362x; 1.0013x over previous
import jax
import jax.numpy as jnp
from jax.experimental import pallas as pl
from jax.experimental.pallas import tpu as pltpu

_BS = 1024


def _body(pe_ref, x_ref, o_ref):
    o_ref[...] = x_ref[...] + pe_ref[...]


def kernel(x, abs_pe):
    B, S, D = x.shape
    nsb = S // _BS
    x2 = x.reshape(B * S, D)
    pe2 = abs_pe.reshape(abs_pe.shape[1], D)
    grid = (nsb, B)
    out = pl.pallas_call(
        _body,
        grid=grid,
        in_specs=[
            pl.BlockSpec((_BS, D), lambda s, b: (s, 0)),
            pl.BlockSpec((_BS, D), lambda s, b: (b * nsb + s, 0)),
        ],
        out_specs=pl.BlockSpec((_BS, D), lambda s, b: (b * nsb + s, 0)),
        out_shape=jax.ShapeDtypeStruct((B * S, D), x.dtype),
        compiler_params=pltpu.CompilerParams(
            dimension_semantics=("arbitrary", "arbitrary"),
        ),
    )(pe2, x2)
    return out.reshape(B, S, D)
